# baseline (speedup 1.0000x reference)
import functools

import jax
import jax.numpy as jnp
from jax import lax
from jax.experimental import pallas as pl
from jax.experimental.pallas import tpu as pltpu
from jax.experimental.pallas import tpu_sc as plsc

N = 10000
E = 320000
D = 128

NC = 2
NS = 16
CHUNK = 128
NW = NC * NS
NCH = 78
TAILBASE = NW * NCH * CHUNK
NTAIL = (E - TAILBASE) // CHUNK
NROWS = N
ROWS_MAIN = 624
TAIL = 16
ZROWS = 104


def _sc_aggregate_body(sd_hbm, h_hbm, out_hbm,
                       sd_v0, sd_v1, rows_v0, rows_v1,
                       zero_v, agg_sh, sem0, sem1, ssem0, ssem1,
                       isem0, isem1):
    c = lax.axis_index("c")
    s = lax.axis_index("s")
    wid = c * NS + s
    sd_v = (sd_v0, sd_v1)
    rows_v = (rows_v0, rows_v1)
    sems = (sem0, sem1)
    ssems = (ssem0, ssem1)
    isems = (isem0, isem1)

    zvec = jnp.zeros((16,), jnp.float32)

    @pl.loop(0, ZROWS)
    def _zero_fill(i):
        for j in range(D // 16):
            zero_v[i, pl.ds(j * 16, 16)] = zvec

    @pl.loop(0, ROWS_MAIN // ZROWS)
    def _zero_agg(j):
        pltpu.sync_copy(zero_v, agg_sh.at[pl.ds(s * ROWS_MAIN + j * ZROWS, ZROWS)])

    @pl.when(s == NS - 1)
    def _zero_tail():
        pltpu.sync_copy(zero_v.at[pl.ds(0, TAIL)], agg_sh.at[pl.ds(NS * ROWS_MAIN, TAIL)])

    plsc.subcore_barrier()

    base = wid * NCH

    def _idx_start(j, b):
        pltpu.async_copy(sd_hbm.at[j], sd_v[b], isems[b])

    def _idx_wait(j, b):
        pltpu.make_async_copy(sd_hbm.at[j], sd_v[b], isems[b]).wait()

    def _load_idx(j, b):
        pltpu.sync_copy(sd_hbm.at[j], sd_v[b])

    def _gather_start(b):
        pltpu.async_copy(h_hbm.at[sd_v[b].at[0]], rows_v[b], sems[b])

    def _gather_wait(b):
        pltpu.make_async_copy(h_hbm.at[sd_v[b].at[0]], rows_v[b],
                              sems[b]).wait()

    def _scatter_start(b):
        pltpu.async_copy(rows_v[b], agg_sh.at[sd_v[b].at[1]], ssems[b],
                         add=True)

    def _scatter_wait(b):
        pltpu.make_async_copy(rows_v[b], agg_sh.at[sd_v[b].at[1]],
                              ssems[b]).wait()

    _load_idx(base, 0)
    _gather_start(0)

    _load_idx(base + 1, 1)
    _gather_start(1)
    _gather_wait(0)
    _scatter_start(0)

    @pl.loop(1, NCH - 1, step=2)
    def _edges(i):
        for b2 in range(2):
            j = i + b2
            b = (1 + b2) % 2
            nb = 1 - b
            _scatter_wait(nb)
            _idx_start(base + j + 1, nb)
            _gather_wait(b)
            _scatter_start(b)
            _idx_wait(base + j + 1, nb)
            _gather_start(nb)

    _scatter_wait(0)
    _gather_wait(1)
    _scatter_start(1)
    _scatter_wait(1)

    @pl.when(wid < NTAIL)
    def _tail():
        off = TAILBASE // CHUNK + wid
        _load_idx(off, 0)
        _gather_start(0)
        _gather_wait(0)
        pltpu.sync_copy(rows_v[0], agg_sh.at[sd_v[0].at[1]], add=True)

    plsc.subcore_barrier()

    pltpu.sync_copy(agg_sh.at[pl.ds(s * ROWS_MAIN, ROWS_MAIN)],
                    out_hbm.at[c, pl.ds(s * ROWS_MAIN, ROWS_MAIN)])

    @pl.when(s == NS - 1)
    def _flush_tail():
        pltpu.sync_copy(agg_sh.at[pl.ds(NS * ROWS_MAIN, TAIL)],
                        out_hbm.at[c, pl.ds(NS * ROWS_MAIN, TAIL)])


@jax.jit
def _sc_aggregate(sd, h):
    mesh = plsc.VectorSubcoreMesh(core_axis_name="c", subcore_axis_name="s")
    return pl.kernel(
        _sc_aggregate_body,
        out_type=jax.ShapeDtypeStruct((NC, N, D), jnp.float32),
        mesh=mesh,
        scratch_types=[
            pltpu.VMEM((2, CHUNK), jnp.int32),
            pltpu.VMEM((2, CHUNK), jnp.int32),
            pltpu.VMEM((CHUNK, D), jnp.float32),
            pltpu.VMEM((CHUNK, D), jnp.float32),
            pltpu.VMEM((ZROWS, D), jnp.float32),
            pltpu.VMEM_SHARED((NROWS, D), jnp.float32),
            pltpu.SemaphoreType.DMA,
            pltpu.SemaphoreType.DMA,
            pltpu.SemaphoreType.DMA,
            pltpu.SemaphoreType.DMA,
            pltpu.SemaphoreType.DMA,
            pltpu.SemaphoreType.DMA,
        ],
    )(sd, h)


BN = 1000


def _mlp_body(h_ref, a_ref, w1_ref, b1_ref, w2_ref, b2_ref, out_ref):
    t = h_ref[...] + a_ref[0] + a_ref[1]
    t = jnp.maximum(jnp.dot(t, w1_ref[...], preferred_element_type=jnp.float32)
                    + b1_ref[...], 0.0)
    t = jnp.dot(t, w2_ref[...], preferred_element_type=jnp.float32) + b2_ref[...]
    out_ref[...] = jnp.maximum(t, 0.0)


def _mlp_final_body(h_ref, a_ref, w1_ref, b1_ref, w2_ref, b2_ref,
                    wl_ref, bl_ref, out_ref):
    t = h_ref[...] + a_ref[0] + a_ref[1]
    t = jnp.maximum(jnp.dot(t, w1_ref[...], preferred_element_type=jnp.float32)
                    + b1_ref[...], 0.0)
    t = jnp.dot(t, w2_ref[...], preferred_element_type=jnp.float32) + b2_ref[...]
    t = jnp.maximum(t, 0.0)
    out_ref[...] = jnp.dot(t, wl_ref[...], preferred_element_type=jnp.float32) + bl_ref[...]


_row_spec = pl.BlockSpec((BN, D), lambda i: (i, 0))
_agg_spec = pl.BlockSpec((NC, BN, D), lambda i: (0, i, 0))
_w_spec = pl.BlockSpec((D, D), lambda i: (0, 0))
_b_spec = pl.BlockSpec((1, D), lambda i: (0, 0))


@jax.jit
def _mlp(h, agg, w1, b1, w2, b2):
    return pl.pallas_call(
        _mlp_body,
        grid=(N // BN,),
        in_specs=[_row_spec, _agg_spec, _w_spec, _b_spec, _w_spec, _b_spec],
        out_specs=_row_spec,
        out_shape=jax.ShapeDtypeStruct((N, D), jnp.float32),
    )(h, agg, w1, b1.reshape(1, D), w2, b2.reshape(1, D))


@jax.jit
def _mlp_final(h, agg, w1, b1, w2, b2, wl, bl):
    return pl.pallas_call(
        _mlp_final_body,
        grid=(N // BN,),
        in_specs=[_row_spec, _agg_spec, _w_spec, _b_spec, _w_spec, _b_spec,
                  _w_spec, _b_spec],
        out_specs=_row_spec,
        out_shape=jax.ShapeDtypeStruct((N, D), jnp.float32),
    )(h, agg, w1, b1.reshape(1, D), w2, b2.reshape(1, D),
      wl, bl.reshape(1, D))


def kernel(x, edge_index, W1_0, b1_0, W2_0, b2_0, W1_1, b1_1, W2_1, b2_1,
           W1_2, b1_2, W2_2, b2_2, Wlin, blin):
    sd = jnp.stack([edge_index[0].reshape(E // CHUNK, CHUNK),
                    edge_index[1].reshape(E // CHUNK, CHUNK)], axis=1)
    agg0 = _sc_aggregate(sd, x)
    h1 = _mlp(x, agg0, W1_0, b1_0, W2_0, b2_0)
    agg1 = _sc_aggregate(sd, h1)
    h2 = _mlp(h1, agg1, W1_1, b1_1, W2_1, b2_1)
    agg2 = _sc_aggregate(sd, h2)
    return _mlp_final(h2, agg2, W1_2, b1_2, W2_2, b2_2, Wlin, blin)


# speedup vs baseline: 1.0478x; 1.0478x over previous
import functools

import jax
import jax.numpy as jnp
from jax import lax
from jax.experimental import pallas as pl
from jax.experimental.pallas import tpu as pltpu
from jax.experimental.pallas import tpu_sc as plsc

N = 10000
E = 320000
D = 128

NC = 2
NS = 16
CHUNK = 128
NW = NC * NS
NCH = 78
TAILBASE = NW * NCH * CHUNK
NTAIL = (E - TAILBASE) // CHUNK
NROWS = N
ROWS_MAIN = 624
TAIL = 16
ZROWS = 104


def _sc_aggregate_body(sd_hbm, h_hbm, out_hbm,
                       sd_v0, sd_v1, rows_v0, rows_v1,
                       zero_v, agg_sh, sem0, sem1, ssem0, ssem1):
    c = lax.axis_index("c")
    s = lax.axis_index("s")
    wid = c * NS + s
    sd_v = (sd_v0, sd_v1)
    rows_v = (rows_v0, rows_v1)
    sems = (sem0, sem1)
    ssems = (ssem0, ssem1)

    zvec = jnp.zeros((16,), jnp.float32)

    @pl.loop(0, ZROWS)
    def _zero_fill(i):
        for j in range(D // 16):
            zero_v[i, pl.ds(j * 16, 16)] = zvec

    @pl.loop(0, ROWS_MAIN // ZROWS)
    def _zero_agg(j):
        pltpu.sync_copy(zero_v, agg_sh.at[pl.ds(s * ROWS_MAIN + j * ZROWS, ZROWS)])

    @pl.when(s == NS - 1)
    def _zero_tail():
        pltpu.sync_copy(zero_v.at[pl.ds(0, TAIL)], agg_sh.at[pl.ds(NS * ROWS_MAIN, TAIL)])

    plsc.subcore_barrier()

    base = wid * NCH

    def _load_idx(j, b):
        pltpu.sync_copy(sd_hbm.at[j], sd_v[b])

    def _gather_start(b):
        pltpu.async_copy(h_hbm.at[sd_v[b].at[0]], rows_v[b], sems[b])

    def _gather_wait(b):
        pltpu.make_async_copy(h_hbm.at[sd_v[b].at[0]], rows_v[b],
                              sems[b]).wait()

    def _scatter_start(b):
        pltpu.async_copy(rows_v[b], agg_sh.at[sd_v[b].at[1]], ssems[b],
                         add=True)

    def _scatter_wait(b):
        pltpu.make_async_copy(rows_v[b], agg_sh.at[sd_v[b].at[1]],
                              ssems[b]).wait()

    _load_idx(base, 0)
    _gather_start(0)

    _load_idx(base + 1, 1)
    _gather_start(1)
    _gather_wait(0)
    _scatter_start(0)

    @pl.loop(1, NCH - 1, step=2)
    def _edges(i):
        for b2 in range(2):
            j = i + b2
            b = (1 + b2) % 2
            nb = 1 - b
            _scatter_wait(nb)
            _load_idx(base + j + 1, nb)
            _gather_start(nb)
            _gather_wait(b)
            _scatter_start(b)

    _scatter_wait(0)
    _gather_wait(1)
    _scatter_start(1)
    _scatter_wait(1)

    @pl.when(wid < NTAIL)
    def _tail():
        off = TAILBASE // CHUNK + wid
        _load_idx(off, 0)
        _gather_start(0)
        _gather_wait(0)
        pltpu.sync_copy(rows_v[0], agg_sh.at[sd_v[0].at[1]], add=True)

    plsc.subcore_barrier()

    pltpu.sync_copy(agg_sh.at[pl.ds(s * ROWS_MAIN, ROWS_MAIN)],
                    out_hbm.at[c, pl.ds(s * ROWS_MAIN, ROWS_MAIN)])

    @pl.when(s == NS - 1)
    def _flush_tail():
        pltpu.sync_copy(agg_sh.at[pl.ds(NS * ROWS_MAIN, TAIL)],
                        out_hbm.at[c, pl.ds(NS * ROWS_MAIN, TAIL)])


@jax.jit
def _sc_aggregate(sd, h):
    mesh = plsc.VectorSubcoreMesh(core_axis_name="c", subcore_axis_name="s")
    return pl.kernel(
        _sc_aggregate_body,
        out_type=jax.ShapeDtypeStruct((NC, N, D), jnp.float32),
        mesh=mesh,
        scratch_types=[
            pltpu.VMEM((2, CHUNK), jnp.int32),
            pltpu.VMEM((2, CHUNK), jnp.int32),
            pltpu.VMEM((CHUNK, D), jnp.float32),
            pltpu.VMEM((CHUNK, D), jnp.float32),
            pltpu.VMEM((ZROWS, D), jnp.float32),
            pltpu.VMEM_SHARED((NROWS, D), jnp.float32),
            pltpu.SemaphoreType.DMA,
            pltpu.SemaphoreType.DMA,
            pltpu.SemaphoreType.DMA,
            pltpu.SemaphoreType.DMA,
        ],
    )(sd, h)


BN = 1000


def _mlp_body(h_ref, a_ref, w1_ref, b1_ref, w2_ref, b2_ref, out_ref):
    t = h_ref[...] + a_ref[0] + a_ref[1]
    t = jnp.maximum(jnp.dot(t, w1_ref[...], preferred_element_type=jnp.float32)
                    + b1_ref[...], 0.0)
    t = jnp.dot(t, w2_ref[...], preferred_element_type=jnp.float32) + b2_ref[...]
    out_ref[...] = jnp.maximum(t, 0.0)


def _mlp_final_body(h_ref, a_ref, w1_ref, b1_ref, w2_ref, b2_ref,
                    wl_ref, bl_ref, out_ref):
    t = h_ref[...] + a_ref[0] + a_ref[1]
    t = jnp.maximum(jnp.dot(t, w1_ref[...], preferred_element_type=jnp.float32)
                    + b1_ref[...], 0.0)
    t = jnp.dot(t, w2_ref[...], preferred_element_type=jnp.float32) + b2_ref[...]
    t = jnp.maximum(t, 0.0)
    out_ref[...] = jnp.dot(t, wl_ref[...], preferred_element_type=jnp.float32) + bl_ref[...]


_row_spec = pl.BlockSpec((BN, D), lambda i: (i, 0))
_agg_spec = pl.BlockSpec((NC, BN, D), lambda i: (0, i, 0))
_w_spec = pl.BlockSpec((D, D), lambda i: (0, 0))
_b_spec = pl.BlockSpec((1, D), lambda i: (0, 0))


@jax.jit
def _mlp(h, agg, w1, b1, w2, b2):
    return pl.pallas_call(
        _mlp_body,
        grid=(N // BN,),
        in_specs=[_row_spec, _agg_spec, _w_spec, _b_spec, _w_spec, _b_spec],
        out_specs=_row_spec,
        out_shape=jax.ShapeDtypeStruct((N, D), jnp.float32),
    )(h, agg, w1, b1.reshape(1, D), w2, b2.reshape(1, D))


@jax.jit
def _mlp_final(h, agg, w1, b1, w2, b2, wl, bl):
    return pl.pallas_call(
        _mlp_final_body,
        grid=(N // BN,),
        in_specs=[_row_spec, _agg_spec, _w_spec, _b_spec, _w_spec, _b_spec,
                  _w_spec, _b_spec],
        out_specs=_row_spec,
        out_shape=jax.ShapeDtypeStruct((N, D), jnp.float32),
    )(h, agg, w1, b1.reshape(1, D), w2, b2.reshape(1, D),
      wl, bl.reshape(1, D))


def kernel(x, edge_index, W1_0, b1_0, W2_0, b2_0, W1_1, b1_1, W2_1, b2_1,
           W1_2, b1_2, W2_2, b2_2, Wlin, blin):
    sd = jnp.stack([edge_index[0].reshape(E // CHUNK, CHUNK),
                    edge_index[1].reshape(E // CHUNK, CHUNK)], axis=1)
    agg0 = _sc_aggregate(sd, x)
    h1 = _mlp(x, agg0, W1_0, b1_0, W2_0, b2_0)
    agg1 = _sc_aggregate(sd, h1)
    h2 = _mlp(h1, agg1, W1_1, b1_1, W2_1, b2_1)
    agg2 = _sc_aggregate(sd, h2)
    return _mlp_final(h2, agg2, W1_2, b1_2, W2_2, b2_2, Wlin, blin)


# speedup vs baseline: 1.0871x; 1.0376x over previous
import functools

import jax
import jax.numpy as jnp
from jax import lax
from jax.experimental import pallas as pl
from jax.experimental.pallas import tpu as pltpu
from jax.experimental.pallas import tpu_sc as plsc

N = 10000
E = 320000
D = 128

NC = 2
NS = 16
CHUNK = 128
NW = NC * NS
NCH = 78
TAILBASE = NW * NCH * CHUNK
NTAIL = (E - TAILBASE) // CHUNK
NROWS = N
ROWS_MAIN = 624
TAIL = 16
ZROWS = 104


def _sc_aggregate_body(sd_hbm, h_hbm, out_hbm,
                       sd_v0, sd_v1, sd_v2, rows_v0, rows_v1, rows_v2,
                       agg_sh, sem0, sem1, sem2, ssem0, ssem1, ssem2):
    c = lax.axis_index("c")
    s = lax.axis_index("s")
    wid = c * NS + s
    sd_v = (sd_v0, sd_v1, sd_v2)
    rows_v = (rows_v0, rows_v1, rows_v2)
    sems = (sem0, sem1, sem2)
    ssems = (ssem0, ssem1, ssem2)

    zvec = jnp.zeros((16,), jnp.float32)

    @pl.loop(0, CHUNK)
    def _zero_fill(i):
        for j in range(D // 16):
            rows_v0[i, pl.ds(j * 16, 16)] = zvec

    for j in range(4):
        pltpu.sync_copy(rows_v0, agg_sh.at[pl.ds(s * ROWS_MAIN + j * CHUNK, CHUNK)])
    pltpu.sync_copy(rows_v0.at[pl.ds(0, ROWS_MAIN - 4 * CHUNK)],
                    agg_sh.at[pl.ds(s * ROWS_MAIN + 4 * CHUNK,
                                    ROWS_MAIN - 4 * CHUNK)])

    @pl.when(s == NS - 1)
    def _zero_tail():
        pltpu.sync_copy(rows_v0.at[pl.ds(0, TAIL)],
                        agg_sh.at[pl.ds(NS * ROWS_MAIN, TAIL)])

    plsc.subcore_barrier()

    base = wid * NCH

    def _load_idx(j, b):
        pltpu.sync_copy(sd_hbm.at[j], sd_v[b])

    def _gather_start(b):
        pltpu.async_copy(h_hbm.at[sd_v[b].at[0]], rows_v[b], sems[b])

    def _gather_wait(b):
        pltpu.make_async_copy(h_hbm.at[sd_v[b].at[0]], rows_v[b],
                              sems[b]).wait()

    def _scatter_start(b):
        pltpu.async_copy(rows_v[b], agg_sh.at[sd_v[b].at[1]], ssems[b],
                         add=True)

    def _scatter_wait(b):
        pltpu.make_async_copy(rows_v[b], agg_sh.at[sd_v[b].at[1]],
                              ssems[b]).wait()

    def _step(j, b, first=False, prefetch=True):
        bp = (b + 2) % 3
        if not first:
            _scatter_wait(bp)
        if prefetch:
            _load_idx(base + j + 2, bp)
            _gather_start(bp)
        _gather_wait(b)
        _scatter_start(b)

    _load_idx(base, 0)
    _gather_start(0)
    _load_idx(base + 1, 1)
    _gather_start(1)

    _step(0, 0, first=True)
    _step(1, 1)

    @pl.loop(2, 74, step=3)
    def _edges(i):
        for b3 in range(3):
            _step(i + b3, (2 + b3) % 3)

    _step(74, 74 % 3)
    _step(75, 75 % 3)
    _step(76, 76 % 3, prefetch=False)
    _step(77, 77 % 3, prefetch=False)
    _scatter_wait(77 % 3)

    @pl.when(wid < NTAIL)
    def _tail():
        off = TAILBASE // CHUNK + wid
        _load_idx(off, 0)
        _gather_start(0)
        _gather_wait(0)
        pltpu.sync_copy(rows_v[0], agg_sh.at[sd_v[0].at[1]], add=True)

    plsc.subcore_barrier()

    pltpu.sync_copy(agg_sh.at[pl.ds(s * ROWS_MAIN, ROWS_MAIN)],
                    out_hbm.at[c, pl.ds(s * ROWS_MAIN, ROWS_MAIN)])

    @pl.when(s == NS - 1)
    def _flush_tail():
        pltpu.sync_copy(agg_sh.at[pl.ds(NS * ROWS_MAIN, TAIL)],
                        out_hbm.at[c, pl.ds(NS * ROWS_MAIN, TAIL)])


@jax.jit
def _sc_aggregate(sd, h):
    mesh = plsc.VectorSubcoreMesh(core_axis_name="c", subcore_axis_name="s")
    return pl.kernel(
        _sc_aggregate_body,
        out_type=jax.ShapeDtypeStruct((NC, N, D), jnp.float32),
        mesh=mesh,
        scratch_types=[
            pltpu.VMEM((2, CHUNK), jnp.int32),
            pltpu.VMEM((2, CHUNK), jnp.int32),
            pltpu.VMEM((2, CHUNK), jnp.int32),
            pltpu.VMEM((CHUNK, D), jnp.float32),
            pltpu.VMEM((CHUNK, D), jnp.float32),
            pltpu.VMEM((CHUNK, D), jnp.float32),
            pltpu.VMEM_SHARED((NROWS, D), jnp.float32),
            pltpu.SemaphoreType.DMA,
            pltpu.SemaphoreType.DMA,
            pltpu.SemaphoreType.DMA,
            pltpu.SemaphoreType.DMA,
            pltpu.SemaphoreType.DMA,
            pltpu.SemaphoreType.DMA,
        ],
    )(sd, h)


BN = 1000


def _mlp_body(h_ref, a_ref, w1_ref, b1_ref, w2_ref, b2_ref, out_ref):
    t = h_ref[...] + a_ref[0] + a_ref[1]
    t = jnp.maximum(jnp.dot(t, w1_ref[...], preferred_element_type=jnp.float32)
                    + b1_ref[...], 0.0)
    t = jnp.dot(t, w2_ref[...], preferred_element_type=jnp.float32) + b2_ref[...]
    out_ref[...] = jnp.maximum(t, 0.0)


def _mlp_final_body(h_ref, a_ref, w1_ref, b1_ref, w2_ref, b2_ref,
                    wl_ref, bl_ref, out_ref):
    t = h_ref[...] + a_ref[0] + a_ref[1]
    t = jnp.maximum(jnp.dot(t, w1_ref[...], preferred_element_type=jnp.float32)
                    + b1_ref[...], 0.0)
    t = jnp.dot(t, w2_ref[...], preferred_element_type=jnp.float32) + b2_ref[...]
    t = jnp.maximum(t, 0.0)
    out_ref[...] = jnp.dot(t, wl_ref[...], preferred_element_type=jnp.float32) + bl_ref[...]


_row_spec = pl.BlockSpec((BN, D), lambda i: (i, 0))
_agg_spec = pl.BlockSpec((NC, BN, D), lambda i: (0, i, 0))
_w_spec = pl.BlockSpec((D, D), lambda i: (0, 0))
_b_spec = pl.BlockSpec((1, D), lambda i: (0, 0))


@jax.jit
def _mlp(h, agg, w1, b1, w2, b2):
    return pl.pallas_call(
        _mlp_body,
        grid=(N // BN,),
        in_specs=[_row_spec, _agg_spec, _w_spec, _b_spec, _w_spec, _b_spec],
        out_specs=_row_spec,
        out_shape=jax.ShapeDtypeStruct((N, D), jnp.float32),
    )(h, agg, w1, b1.reshape(1, D), w2, b2.reshape(1, D))


@jax.jit
def _mlp_final(h, agg, w1, b1, w2, b2, wl, bl):
    return pl.pallas_call(
        _mlp_final_body,
        grid=(N // BN,),
        in_specs=[_row_spec, _agg_spec, _w_spec, _b_spec, _w_spec, _b_spec,
                  _w_spec, _b_spec],
        out_specs=_row_spec,
        out_shape=jax.ShapeDtypeStruct((N, D), jnp.float32),
    )(h, agg, w1, b1.reshape(1, D), w2, b2.reshape(1, D),
      wl, bl.reshape(1, D))


def kernel(x, edge_index, W1_0, b1_0, W2_0, b2_0, W1_1, b1_1, W2_1, b2_1,
           W1_2, b1_2, W2_2, b2_2, Wlin, blin):
    sd = jnp.stack([edge_index[0].reshape(E // CHUNK, CHUNK),
                    edge_index[1].reshape(E // CHUNK, CHUNK)], axis=1)
    agg0 = _sc_aggregate(sd, x)
    h1 = _mlp(x, agg0, W1_0, b1_0, W2_0, b2_0)
    agg1 = _sc_aggregate(sd, h1)
    h2 = _mlp(h1, agg1, W1_1, b1_1, W2_1, b2_1)
    agg2 = _sc_aggregate(sd, h2)
    return _mlp_final(h2, agg2, W1_2, b1_2, W2_2, b2_2, Wlin, blin)


# speedup vs baseline: 1.1777x; 1.0833x over previous
import functools

import jax
import jax.numpy as jnp
from jax import lax
from jax.experimental import pallas as pl
from jax.experimental.pallas import tpu as pltpu
from jax.experimental.pallas import tpu_sc as plsc

N = 10000
E = 320000
D = 128

NC = 2
NS = 16
CHUNK = 128
NW = NC * NS
NCH = 78
TAILBASE = NW * NCH * CHUNK
NTAIL = (E - TAILBASE) // CHUNK
NROWS = N
ROWS_MAIN = 624
TAIL = 16
ZROWS = 104


def _sc_aggregate_body(sd_hbm, h_hbm, out_hbm,
                       sd_v0, sd_v1, sd_v2, rows_v0, rows_v1, rows_v2,
                       agg_sh, sem0, sem1, sem2, ssem0, ssem1, ssem2):
    c = lax.axis_index("c")
    s = lax.axis_index("s")
    wid = c * NS + s
    sd_v = (sd_v0, sd_v1, sd_v2)
    rows_v = (rows_v0, rows_v1, rows_v2)
    sems = (sem0, sem1, sem2)
    ssems = (ssem0, ssem1, ssem2)

    zvec = jnp.zeros((16,), jnp.float32)

    @pl.loop(0, CHUNK)
    def _zero_fill(i):
        for j in range(D // 16):
            rows_v0[i, pl.ds(j * 16, 16)] = zvec

    for j in range(4):
        pltpu.sync_copy(rows_v0, agg_sh.at[pl.ds(s * ROWS_MAIN + j * CHUNK, CHUNK)])
    pltpu.sync_copy(rows_v0.at[pl.ds(0, ROWS_MAIN - 4 * CHUNK)],
                    agg_sh.at[pl.ds(s * ROWS_MAIN + 4 * CHUNK,
                                    ROWS_MAIN - 4 * CHUNK)])

    @pl.when(s == NS - 1)
    def _zero_tail():
        pltpu.sync_copy(rows_v0.at[pl.ds(0, TAIL)],
                        agg_sh.at[pl.ds(NS * ROWS_MAIN, TAIL)])

    plsc.subcore_barrier()

    base = wid * NCH

    def _load_idx(j, b):
        pltpu.sync_copy(sd_hbm.at[j], sd_v[b])

    def _gather_start(b):
        pltpu.async_copy(h_hbm.at[sd_v[b].at[0]], rows_v[b], sems[b])

    def _gather_wait(b):
        pltpu.make_async_copy(h_hbm.at[sd_v[b].at[0]], rows_v[b],
                              sems[b]).wait()

    def _scatter_start(b):
        pltpu.async_copy(rows_v[b], agg_sh.at[sd_v[b].at[1]], ssems[b],
                         add=True)

    def _scatter_wait(b):
        pltpu.make_async_copy(rows_v[b], agg_sh.at[sd_v[b].at[1]],
                              ssems[b]).wait()

    def _step(j, b, first=False, prefetch=True):
        bp = (b + 2) % 3
        _gather_wait(b)
        _scatter_start(b)
        if not first:
            _scatter_wait(bp)
        if prefetch:
            _load_idx(base + j + 2, bp)
            _gather_start(bp)

    _load_idx(base, 0)
    _gather_start(0)
    _load_idx(base + 1, 1)
    _gather_start(1)

    _step(0, 0, first=True)
    _step(1, 1)

    @pl.loop(2, 74, step=3)
    def _edges(i):
        for b3 in range(3):
            _step(i + b3, (2 + b3) % 3)

    _step(74, 74 % 3)
    _step(75, 75 % 3)
    _step(76, 76 % 3, prefetch=False)
    _step(77, 77 % 3, prefetch=False)
    _scatter_wait(77 % 3)

    @pl.when(wid < NTAIL)
    def _tail():
        off = TAILBASE // CHUNK + wid
        _load_idx(off, 0)
        _gather_start(0)
        _gather_wait(0)
        pltpu.sync_copy(rows_v[0], agg_sh.at[sd_v[0].at[1]], add=True)

    plsc.subcore_barrier()

    pltpu.sync_copy(agg_sh.at[pl.ds(s * ROWS_MAIN, ROWS_MAIN)],
                    out_hbm.at[c, pl.ds(s * ROWS_MAIN, ROWS_MAIN)])

    @pl.when(s == NS - 1)
    def _flush_tail():
        pltpu.sync_copy(agg_sh.at[pl.ds(NS * ROWS_MAIN, TAIL)],
                        out_hbm.at[c, pl.ds(NS * ROWS_MAIN, TAIL)])


@jax.jit
def _sc_aggregate(sd, h):
    mesh = plsc.VectorSubcoreMesh(core_axis_name="c", subcore_axis_name="s")
    return pl.kernel(
        _sc_aggregate_body,
        out_type=jax.ShapeDtypeStruct((NC, N, D), jnp.float32),
        mesh=mesh,
        scratch_types=[
            pltpu.VMEM((2, CHUNK), jnp.int32),
            pltpu.VMEM((2, CHUNK), jnp.int32),
            pltpu.VMEM((2, CHUNK), jnp.int32),
            pltpu.VMEM((CHUNK, D), jnp.float32),
            pltpu.VMEM((CHUNK, D), jnp.float32),
            pltpu.VMEM((CHUNK, D), jnp.float32),
            pltpu.VMEM_SHARED((NROWS, D), jnp.float32),
            pltpu.SemaphoreType.DMA,
            pltpu.SemaphoreType.DMA,
            pltpu.SemaphoreType.DMA,
            pltpu.SemaphoreType.DMA,
            pltpu.SemaphoreType.DMA,
            pltpu.SemaphoreType.DMA,
        ],
    )(sd, h)


BN = 1000


def _mlp_body(h_ref, a_ref, w1_ref, b1_ref, w2_ref, b2_ref, out_ref):
    t = h_ref[...] + a_ref[0] + a_ref[1]
    t = jnp.maximum(jnp.dot(t, w1_ref[...], preferred_element_type=jnp.float32)
                    + b1_ref[...], 0.0)
    t = jnp.dot(t, w2_ref[...], preferred_element_type=jnp.float32) + b2_ref[...]
    out_ref[...] = jnp.maximum(t, 0.0)


def _mlp_final_body(h_ref, a_ref, w1_ref, b1_ref, w2_ref, b2_ref,
                    wl_ref, bl_ref, out_ref):
    t = h_ref[...] + a_ref[0] + a_ref[1]
    t = jnp.maximum(jnp.dot(t, w1_ref[...], preferred_element_type=jnp.float32)
                    + b1_ref[...], 0.0)
    t = jnp.dot(t, w2_ref[...], preferred_element_type=jnp.float32) + b2_ref[...]
    t = jnp.maximum(t, 0.0)
    out_ref[...] = jnp.dot(t, wl_ref[...], preferred_element_type=jnp.float32) + bl_ref[...]


_row_spec = pl.BlockSpec((BN, D), lambda i: (i, 0))
_agg_spec = pl.BlockSpec((NC, BN, D), lambda i: (0, i, 0))
_w_spec = pl.BlockSpec((D, D), lambda i: (0, 0))
_b_spec = pl.BlockSpec((1, D), lambda i: (0, 0))


@jax.jit
def _mlp(h, agg, w1, b1, w2, b2):
    return pl.pallas_call(
        _mlp_body,
        grid=(N // BN,),
        in_specs=[_row_spec, _agg_spec, _w_spec, _b_spec, _w_spec, _b_spec],
        out_specs=_row_spec,
        out_shape=jax.ShapeDtypeStruct((N, D), jnp.float32),
    )(h, agg, w1, b1.reshape(1, D), w2, b2.reshape(1, D))


@jax.jit
def _mlp_final(h, agg, w1, b1, w2, b2, wl, bl):
    return pl.pallas_call(
        _mlp_final_body,
        grid=(N // BN,),
        in_specs=[_row_spec, _agg_spec, _w_spec, _b_spec, _w_spec, _b_spec,
                  _w_spec, _b_spec],
        out_specs=_row_spec,
        out_shape=jax.ShapeDtypeStruct((N, D), jnp.float32),
    )(h, agg, w1, b1.reshape(1, D), w2, b2.reshape(1, D),
      wl, bl.reshape(1, D))


def kernel(x, edge_index, W1_0, b1_0, W2_0, b2_0, W1_1, b1_1, W2_1, b2_1,
           W1_2, b1_2, W2_2, b2_2, Wlin, blin):
    sd = jnp.stack([edge_index[0].reshape(E // CHUNK, CHUNK),
                    edge_index[1].reshape(E // CHUNK, CHUNK)], axis=1)
    agg0 = _sc_aggregate(sd, x)
    h1 = _mlp(x, agg0, W1_0, b1_0, W2_0, b2_0)
    agg1 = _sc_aggregate(sd, h1)
    h2 = _mlp(h1, agg1, W1_1, b1_1, W2_1, b2_1)
    agg2 = _sc_aggregate(sd, h2)
    return _mlp_final(h2, agg2, W1_2, b1_2, W2_2, b2_2, Wlin, blin)


# speedup vs baseline: 1.1884x; 1.0091x over previous
import functools

import jax
import jax.numpy as jnp
from jax import lax
from jax.experimental import pallas as pl
from jax.experimental.pallas import tpu as pltpu
from jax.experimental.pallas import tpu_sc as plsc

N = 10000
E = 320000
D = 128

NC = 2
NS = 16
CHUNK = 128
NW = NC * NS
NCH = 78
TAILBASE = NW * NCH * CHUNK
NTAIL = (E - TAILBASE) // CHUNK
NROWS = N
ROWS_MAIN = 624
TAIL = 16
ZROWS = 104


def _sc_aggregate_body(sd_hbm, h_hbm, out_hbm,
                       sd_v0, sd_v1, sd_v2, rows_v0, rows_v1, rows_v2,
                       agg_sh, sem0, sem1, sem2, ssem0, ssem1, ssem2):
    c = lax.axis_index("c")
    s = lax.axis_index("s")
    wid = c * NS + s
    sd_v = (sd_v0, sd_v1, sd_v2)
    rows_v = (rows_v0, rows_v1, rows_v2)
    sems = (sem0, sem1, sem2)
    ssems = (ssem0, ssem1, ssem2)

    zvec = jnp.zeros((16,), jnp.float32)

    @pl.loop(0, CHUNK)
    def _zero_fill(i):
        for j in range(D // 16):
            rows_v2[i, pl.ds(j * 16, 16)] = zvec

    base = wid * NCH

    def _load_idx(j, b):
        pltpu.sync_copy(sd_hbm.at[j], sd_v[b])

    def _gather_start(b):
        pltpu.async_copy(h_hbm.at[sd_v[b].at[0]], rows_v[b], sems[b])

    def _gather_wait(b):
        pltpu.make_async_copy(h_hbm.at[sd_v[b].at[0]], rows_v[b],
                              sems[b]).wait()

    def _scatter_start(b):
        pltpu.async_copy(rows_v[b], agg_sh.at[sd_v[b].at[1]], ssems[b],
                         add=True)

    def _scatter_wait(b):
        pltpu.make_async_copy(rows_v[b], agg_sh.at[sd_v[b].at[1]],
                              ssems[b]).wait()

    def _step(j, b, first=False, prefetch=True):
        bp = (b + 2) % 3
        _gather_wait(b)
        _scatter_start(b)
        if not first:
            _scatter_wait(bp)
        if prefetch:
            _load_idx(base + j + 2, bp)
            _gather_start(bp)

    _load_idx(base, 0)
    _gather_start(0)
    _load_idx(base + 1, 1)
    _gather_start(1)

    for j in range(4):
        pltpu.sync_copy(rows_v2, agg_sh.at[pl.ds(s * ROWS_MAIN + j * CHUNK, CHUNK)])
    pltpu.sync_copy(rows_v2.at[pl.ds(0, ROWS_MAIN - 4 * CHUNK)],
                    agg_sh.at[pl.ds(s * ROWS_MAIN + 4 * CHUNK,
                                    ROWS_MAIN - 4 * CHUNK)])

    @pl.when(s == NS - 1)
    def _zero_tail():
        pltpu.sync_copy(rows_v2.at[pl.ds(0, TAIL)],
                        agg_sh.at[pl.ds(NS * ROWS_MAIN, TAIL)])

    plsc.subcore_barrier()

    _step(0, 0, first=True)
    _step(1, 1)

    @pl.loop(2, 74, step=3)
    def _edges(i):
        for b3 in range(3):
            _step(i + b3, (2 + b3) % 3)

    _step(74, 74 % 3)
    _step(75, 75 % 3)
    _step(76, 76 % 3, prefetch=False)
    _step(77, 77 % 3, prefetch=False)
    _scatter_wait(77 % 3)

    @pl.when(wid < NTAIL)
    def _tail():
        off = TAILBASE // CHUNK + wid
        _load_idx(off, 0)
        _gather_start(0)
        _gather_wait(0)
        pltpu.sync_copy(rows_v[0], agg_sh.at[sd_v[0].at[1]], add=True)

    plsc.subcore_barrier()

    pltpu.sync_copy(agg_sh.at[pl.ds(s * ROWS_MAIN, ROWS_MAIN)],
                    out_hbm.at[c, pl.ds(s * ROWS_MAIN, ROWS_MAIN)])

    @pl.when(s == NS - 1)
    def _flush_tail():
        pltpu.sync_copy(agg_sh.at[pl.ds(NS * ROWS_MAIN, TAIL)],
                        out_hbm.at[c, pl.ds(NS * ROWS_MAIN, TAIL)])


@jax.jit
def _sc_aggregate(sd, h):
    mesh = plsc.VectorSubcoreMesh(core_axis_name="c", subcore_axis_name="s")
    return pl.kernel(
        _sc_aggregate_body,
        out_type=jax.ShapeDtypeStruct((NC, N, D), jnp.float32),
        mesh=mesh,
        scratch_types=[
            pltpu.VMEM((2, CHUNK), jnp.int32),
            pltpu.VMEM((2, CHUNK), jnp.int32),
            pltpu.VMEM((2, CHUNK), jnp.int32),
            pltpu.VMEM((CHUNK, D), jnp.float32),
            pltpu.VMEM((CHUNK, D), jnp.float32),
            pltpu.VMEM((CHUNK, D), jnp.float32),
            pltpu.VMEM_SHARED((NROWS, D), jnp.float32),
            pltpu.SemaphoreType.DMA,
            pltpu.SemaphoreType.DMA,
            pltpu.SemaphoreType.DMA,
            pltpu.SemaphoreType.DMA,
            pltpu.SemaphoreType.DMA,
            pltpu.SemaphoreType.DMA,
        ],
    )(sd, h)


BN = 1000


def _mlp_body(h_ref, a_ref, w1_ref, b1_ref, w2_ref, b2_ref, out_ref):
    t = h_ref[...] + a_ref[0] + a_ref[1]
    t = jnp.maximum(jnp.dot(t, w1_ref[...], preferred_element_type=jnp.float32)
                    + b1_ref[...], 0.0)
    t = jnp.dot(t, w2_ref[...], preferred_element_type=jnp.float32) + b2_ref[...]
    out_ref[...] = jnp.maximum(t, 0.0)


def _mlp_final_body(h_ref, a_ref, w1_ref, b1_ref, w2_ref, b2_ref,
                    wl_ref, bl_ref, out_ref):
    t = h_ref[...] + a_ref[0] + a_ref[1]
    t = jnp.maximum(jnp.dot(t, w1_ref[...], preferred_element_type=jnp.float32)
                    + b1_ref[...], 0.0)
    t = jnp.dot(t, w2_ref[...], preferred_element_type=jnp.float32) + b2_ref[...]
    t = jnp.maximum(t, 0.0)
    out_ref[...] = jnp.dot(t, wl_ref[...], preferred_element_type=jnp.float32) + bl_ref[...]


_row_spec = pl.BlockSpec((BN, D), lambda i: (i, 0))
_agg_spec = pl.BlockSpec((NC, BN, D), lambda i: (0, i, 0))
_w_spec = pl.BlockSpec((D, D), lambda i: (0, 0))
_b_spec = pl.BlockSpec((1, D), lambda i: (0, 0))


@jax.jit
def _mlp(h, agg, w1, b1, w2, b2):
    return pl.pallas_call(
        _mlp_body,
        grid=(N // BN,),
        in_specs=[_row_spec, _agg_spec, _w_spec, _b_spec, _w_spec, _b_spec],
        out_specs=_row_spec,
        out_shape=jax.ShapeDtypeStruct((N, D), jnp.float32),
    )(h, agg, w1, b1.reshape(1, D), w2, b2.reshape(1, D))


@jax.jit
def _mlp_final(h, agg, w1, b1, w2, b2, wl, bl):
    return pl.pallas_call(
        _mlp_final_body,
        grid=(N // BN,),
        in_specs=[_row_spec, _agg_spec, _w_spec, _b_spec, _w_spec, _b_spec,
                  _w_spec, _b_spec],
        out_specs=_row_spec,
        out_shape=jax.ShapeDtypeStruct((N, D), jnp.float32),
    )(h, agg, w1, b1.reshape(1, D), w2, b2.reshape(1, D),
      wl, bl.reshape(1, D))


def kernel(x, edge_index, W1_0, b1_0, W2_0, b2_0, W1_1, b1_1, W2_1, b2_1,
           W1_2, b1_2, W2_2, b2_2, Wlin, blin):
    sd = jnp.stack([edge_index[0].reshape(E // CHUNK, CHUNK),
                    edge_index[1].reshape(E // CHUNK, CHUNK)], axis=1)
    agg0 = _sc_aggregate(sd, x)
    h1 = _mlp(x, agg0, W1_0, b1_0, W2_0, b2_0)
    agg1 = _sc_aggregate(sd, h1)
    h2 = _mlp(h1, agg1, W1_1, b1_1, W2_1, b2_1)
    agg2 = _sc_aggregate(sd, h2)
    return _mlp_final(h2, agg2, W1_2, b1_2, W2_2, b2_2, Wlin, blin)
